# Initial kernel scaffold; baseline (speedup 1.0000x reference)
import jax
import jax.numpy as jnp
from jax.experimental import pallas as pl


def kernel(x, edge_index, W1, b1, W2, b2, W3, b3, Wo, bo):
    raise NotImplementedError("write your pallas kernel here")


# speedup vs baseline: 10.7303x; 10.7303x over previous
import functools

import jax
import jax.numpy as jnp
from jax import lax
from jax.experimental import pallas as pl
from jax.experimental.pallas import tpu as pltpu
from jax.experimental.pallas import tpu_sc as plsc

_N = 10000
_E = 320000
_D = 128
_NC = 2
_NS = 16
_NW = _NC * _NS
_EPW = _E // _NW
_CH = 80
_NCH = _EPW // _CH
_RPT = 640
_NP = _RPT * _NS
_BLK = 1024
_GRID = _NP // _BLK


def _sc_mesh():
    return plsc.VectorSubcoreMesh(core_axis_name="c", subcore_axis_name="s")


@functools.partial(
    pl.kernel,
    out_type=jax.ShapeDtypeStruct((2 * _NP, _D), jnp.float32),
    mesh=_sc_mesh(),
    scratch_types=[
        pltpu.VMEM_SHARED((_NP, _D), jnp.float32),
        pltpu.VMEM((_CH,), jnp.int32),
        pltpu.VMEM((_CH, _D), jnp.float32),
    ],
)
def _sc_deg(dst_hbm, zeros_hbm, ones_hbm, out_hbm, acc, didx, ones_v):
    c = lax.axis_index("c")
    s = lax.axis_index("s")
    wid = s * _NC + c
    base = wid * _EPW
    pltpu.sync_copy(zeros_hbm.at[pl.ds(s * _RPT, _RPT)],
                    acc.at[pl.ds(s * _RPT, _RPT)])
    pltpu.sync_copy(ones_hbm, ones_v)
    plsc.subcore_barrier()

    def step(j, carry):
        off = base + j * _CH
        pltpu.sync_copy(dst_hbm.at[pl.ds(off, _CH)], didx)
        pltpu.sync_copy(ones_v, acc.at[didx], add=True)
        return carry

    lax.fori_loop(0, _NCH, step, 0)
    plsc.subcore_barrier()
    pltpu.sync_copy(acc.at[pl.ds(s * _RPT, _RPT)],
                    out_hbm.at[pl.ds(c * _NP + s * _RPT, _RPT)])


@functools.partial(
    pl.kernel,
    out_type=jax.ShapeDtypeStruct((2 * _NP, _D), jnp.float32),
    mesh=_sc_mesh(),
    scratch_types=[
        pltpu.VMEM_SHARED((_NP, _D), jnp.float32),
        pltpu.VMEM((_CH,), jnp.int32),
        pltpu.VMEM((_CH,), jnp.int32),
        pltpu.VMEM((_CH, _D), jnp.float32),
        pltpu.SemaphoreType.DMA,
    ],
)
def _sc_agg(table_hbm, src_hbm, dst_hbm, zeros_hbm, out_hbm,
            acc, sidx, didx, rows, sem):
    c = lax.axis_index("c")
    s = lax.axis_index("s")
    wid = s * _NC + c
    base = wid * _EPW
    pltpu.sync_copy(zeros_hbm.at[pl.ds(s * _RPT, _RPT)],
                    acc.at[pl.ds(s * _RPT, _RPT)])
    plsc.subcore_barrier()

    def step(j, carry):
        off = base + j * _CH
        pltpu.sync_copy(src_hbm.at[pl.ds(off, _CH)], sidx)
        pltpu.sync_copy(dst_hbm.at[pl.ds(off, _CH)], didx)
        pltpu.async_copy(table_hbm.at[sidx], rows, sem).wait()
        pltpu.sync_copy(rows, acc.at[didx], add=True)
        return carry

    lax.fori_loop(0, _NCH, step, 0)
    plsc.subcore_barrier()
    pltpu.sync_copy(acc.at[pl.ds(s * _RPT, _RPT)],
                    out_hbm.at[pl.ds(c * _NP + s * _RPT, _RPT)])


def _tc_first_body(deg0_ref, deg1_ref, x_ref, w_ref, dis_ref, hwp_ref):
    p0 = deg0_ref[:, 0:1]
    p1 = deg1_ref[:, 0:1]
    dis = lax.rsqrt(1.0 + p0 + p1)
    dis_b = jnp.broadcast_to(dis, (_BLK, _D))
    hw = jnp.dot(x_ref[...], w_ref[...], preferred_element_type=jnp.float32)
    dis_ref[...] = dis_b
    hwp_ref[...] = hw * dis_b


def _tc_first(deg, x, W1):
    return pl.pallas_call(
        _tc_first_body,
        grid=(_GRID,),
        in_specs=[
            pl.BlockSpec((_BLK, _D), lambda j: (j, 0)),
            pl.BlockSpec((_BLK, _D), lambda j: (j + _GRID, 0)),
            pl.BlockSpec((_BLK, _D), lambda j: (j, 0)),
            pl.BlockSpec((_D, _D), lambda j: (0, 0)),
        ],
        out_specs=[
            pl.BlockSpec((_BLK, _D), lambda j: (j, 0)),
            pl.BlockSpec((_BLK, _D), lambda j: (j, 0)),
        ],
        out_shape=[
            jax.ShapeDtypeStruct((_NP, _D), jnp.float32),
            jax.ShapeDtypeStruct((_NP, _D), jnp.float32),
        ],
    )(deg, deg, x, W1)


def _tc_mid_body(s0_ref, s1_ref, hwp_ref, dis_ref, b_ref, w_ref, out_ref):
    h = dis_ref[...] * (s0_ref[...] + s1_ref[...] + hwp_ref[...]) + b_ref[...]
    h = jnp.maximum(h, 0.0)
    out_ref[...] = (
        jnp.dot(h, w_ref[...], preferred_element_type=jnp.float32)
        * dis_ref[...]
    )


def _tc_mid(S, hwp, dis, b, W):
    return pl.pallas_call(
        _tc_mid_body,
        grid=(_GRID,),
        in_specs=[
            pl.BlockSpec((_BLK, _D), lambda j: (j, 0)),
            pl.BlockSpec((_BLK, _D), lambda j: (j + _GRID, 0)),
            pl.BlockSpec((_BLK, _D), lambda j: (j, 0)),
            pl.BlockSpec((_BLK, _D), lambda j: (j, 0)),
            pl.BlockSpec((1, _D), lambda j: (0, 0)),
            pl.BlockSpec((_D, _D), lambda j: (0, 0)),
        ],
        out_specs=pl.BlockSpec((_BLK, _D), lambda j: (j, 0)),
        out_shape=jax.ShapeDtypeStruct((_NP, _D), jnp.float32),
    )(S, S, hwp, dis, b, W)


def _tc_last_body(s0_ref, s1_ref, hwp_ref, dis_ref, b_ref, wo_ref, bo_ref,
                  out_ref):
    h = dis_ref[...] * (s0_ref[...] + s1_ref[...] + hwp_ref[...]) + b_ref[...]
    h = jnp.maximum(h, 0.0)
    z = jnp.dot(h, wo_ref[...], preferred_element_type=jnp.float32) + bo_ref[...]
    out_ref[...] = jax.nn.sigmoid(z)


def _tc_last(S, hwp, dis, b3, Wo, bo):
    return pl.pallas_call(
        _tc_last_body,
        grid=(_GRID,),
        in_specs=[
            pl.BlockSpec((_BLK, _D), lambda j: (j, 0)),
            pl.BlockSpec((_BLK, _D), lambda j: (j + _GRID, 0)),
            pl.BlockSpec((_BLK, _D), lambda j: (j, 0)),
            pl.BlockSpec((_BLK, _D), lambda j: (j, 0)),
            pl.BlockSpec((1, _D), lambda j: (0, 0)),
            pl.BlockSpec((_D, 1), lambda j: (0, 0)),
            pl.BlockSpec((1, 1), lambda j: (0, 0)),
        ],
        out_specs=pl.BlockSpec((_BLK, 1), lambda j: (j, 0)),
        out_shape=jax.ShapeDtypeStruct((_N, 1), jnp.float32),
    )(S, S, hwp, dis, b3, Wo, bo)


def kernel(x, edge_index, W1, b1, W2, b2, W3, b3, Wo, bo):
    src = edge_index[0]
    dst = edge_index[1]
    zeros = jnp.zeros((_NP, _D), jnp.float32)
    ones = jnp.ones((_CH, _D), jnp.float32)

    deg = _sc_deg(dst, zeros, ones)
    dis, hw1p = _tc_first(deg, x, W1)
    S1 = _sc_agg(hw1p, src, dst, zeros)
    hw2p = _tc_mid(S1, hw1p, dis, b1.reshape(1, _D), W2)
    S2 = _sc_agg(hw2p, src, dst, zeros)
    hw3p = _tc_mid(S2, hw2p, dis, b2.reshape(1, _D), W3)
    S3 = _sc_agg(hw3p, src, dst, zeros)
    return _tc_last(S3, hw3p, dis, b3.reshape(1, _D), Wo, bo.reshape(1, 1))


# speedup vs baseline: 19.3670x; 1.8049x over previous
import functools

import jax
import jax.numpy as jnp
from jax import lax
from jax.experimental import pallas as pl
from jax.experimental.pallas import tpu as pltpu
from jax.experimental.pallas import tpu_sc as plsc

_N = 10000
_E = 320000
_D = 128
_NC = 2
_NS = 16
_NW = _NC * _NS
_EPW = _E // _NW
_CH = 80
_NCH = _EPW // _CH
_RPT = 640
_NP = _RPT * _NS
_BLK = 1024
_GRID = _NP // _BLK


def _sc_mesh():
    return plsc.VectorSubcoreMesh(core_axis_name="c", subcore_axis_name="s")


@functools.partial(
    pl.kernel,
    out_type=jax.ShapeDtypeStruct((2 * _NP, _D), jnp.float32),
    mesh=_sc_mesh(),
    scratch_types=[
        pltpu.VMEM_SHARED((_NP, _D), jnp.float32),
        pltpu.VMEM((_NCH, _CH), jnp.int32),
        pltpu.VMEM((_CH, _D), jnp.float32),
    ],
)
def _sc_deg(dst3_hbm, zeros_hbm, ones_hbm, out_hbm, acc, didx, ones_v):
    c = lax.axis_index("c")
    s = lax.axis_index("s")
    wid = s * _NC + c
    pltpu.sync_copy(zeros_hbm.at[pl.ds(s * _RPT, _RPT)],
                    acc.at[pl.ds(s * _RPT, _RPT)])
    pltpu.sync_copy(ones_hbm, ones_v)
    pltpu.sync_copy(dst3_hbm.at[wid], didx)
    plsc.subcore_barrier()

    def step(j, carry):
        pltpu.sync_copy(ones_v, acc.at[didx.at[j]], add=True)
        return carry

    lax.fori_loop(0, _NCH, step, 0)
    plsc.subcore_barrier()
    pltpu.sync_copy(acc.at[pl.ds(s * _RPT, _RPT)],
                    out_hbm.at[pl.ds(c * _NP + s * _RPT, _RPT)])


@functools.partial(
    pl.kernel,
    out_type=jax.ShapeDtypeStruct((2 * _NP, _D), jnp.float32),
    mesh=_sc_mesh(),
    scratch_types=[
        pltpu.VMEM_SHARED((_NP, _D), jnp.float32),
        pltpu.VMEM((_EPW,), jnp.int32),
        pltpu.VMEM((_NCH, _CH), jnp.int32),
        pltpu.VMEM((_CH, _D), jnp.float32),
        pltpu.VMEM((_CH, _D), jnp.float32),
        pltpu.SemaphoreType.DMA,
        pltpu.SemaphoreType.DMA,
    ],
)
def _sc_agg(table_hbm, src_hbm, dst3_hbm, zeros_hbm, out_hbm,
            acc, sidx, didx, rows0, rows1, sem0, sem1):
    c = lax.axis_index("c")
    s = lax.axis_index("s")
    wid = s * _NC + c
    pltpu.sync_copy(zeros_hbm.at[pl.ds(s * _RPT, _RPT)],
                    acc.at[pl.ds(s * _RPT, _RPT)])
    pltpu.sync_copy(src_hbm.at[pl.ds(wid * _EPW, _EPW)], sidx)
    pltpu.sync_copy(dst3_hbm.at[wid], didx)
    plsc.subcore_barrier()

    def sch(j):
        return sidx.at[pl.ds(pl.multiple_of(j * _CH, _CH), _CH)]

    pltpu.async_copy(table_hbm.at[sch(0)], rows0, sem0)

    def pair(i, carry):
        j = 2 * i
        pltpu.make_async_copy(table_hbm.at[sch(j)], rows0, sem0).wait()
        pltpu.async_copy(table_hbm.at[sch(j + 1)], rows1, sem1)
        pltpu.sync_copy(rows0, acc.at[didx.at[j]], add=True)
        pltpu.make_async_copy(table_hbm.at[sch(j + 1)], rows1, sem1).wait()

        @pl.when(j + 2 < _NCH)
        def _():
            pltpu.async_copy(table_hbm.at[sch(j + 2)], rows0, sem0)

        pltpu.sync_copy(rows1, acc.at[didx.at[j + 1]], add=True)
        return carry

    lax.fori_loop(0, (_NCH - 1) // 2, pair, 0)
    pltpu.make_async_copy(table_hbm.at[sch(_NCH - 1)], rows0, sem0).wait()
    pltpu.sync_copy(rows0, acc.at[didx.at[_NCH - 1]], add=True)
    plsc.subcore_barrier()
    pltpu.sync_copy(acc.at[pl.ds(s * _RPT, _RPT)],
                    out_hbm.at[pl.ds(c * _NP + s * _RPT, _RPT)])


def _tc_first_body(deg0_ref, deg1_ref, x_ref, w_ref, dis_ref, hwp_ref):
    p0 = deg0_ref[:, 0:1]
    p1 = deg1_ref[:, 0:1]
    dis = lax.rsqrt(1.0 + p0 + p1)
    dis_b = jnp.broadcast_to(dis, (_BLK, _D))
    hw = jnp.dot(x_ref[...], w_ref[...], preferred_element_type=jnp.float32)
    dis_ref[...] = dis_b
    hwp_ref[...] = hw * dis_b


def _tc_first(deg, x, W1):
    return pl.pallas_call(
        _tc_first_body,
        grid=(_GRID,),
        in_specs=[
            pl.BlockSpec((_BLK, _D), lambda j: (j, 0)),
            pl.BlockSpec((_BLK, _D), lambda j: (j + _GRID, 0)),
            pl.BlockSpec((_BLK, _D), lambda j: (j, 0)),
            pl.BlockSpec((_D, _D), lambda j: (0, 0)),
        ],
        out_specs=[
            pl.BlockSpec((_BLK, _D), lambda j: (j, 0)),
            pl.BlockSpec((_BLK, _D), lambda j: (j, 0)),
        ],
        out_shape=[
            jax.ShapeDtypeStruct((_NP, _D), jnp.float32),
            jax.ShapeDtypeStruct((_NP, _D), jnp.float32),
        ],
    )(deg, deg, x, W1)


def _tc_mid_body(s0_ref, s1_ref, hwp_ref, dis_ref, b_ref, w_ref, out_ref):
    h = dis_ref[...] * (s0_ref[...] + s1_ref[...] + hwp_ref[...]) + b_ref[...]
    h = jnp.maximum(h, 0.0)
    out_ref[...] = (
        jnp.dot(h, w_ref[...], preferred_element_type=jnp.float32)
        * dis_ref[...]
    )


def _tc_mid(S, hwp, dis, b, W):
    return pl.pallas_call(
        _tc_mid_body,
        grid=(_GRID,),
        in_specs=[
            pl.BlockSpec((_BLK, _D), lambda j: (j, 0)),
            pl.BlockSpec((_BLK, _D), lambda j: (j + _GRID, 0)),
            pl.BlockSpec((_BLK, _D), lambda j: (j, 0)),
            pl.BlockSpec((_BLK, _D), lambda j: (j, 0)),
            pl.BlockSpec((1, _D), lambda j: (0, 0)),
            pl.BlockSpec((_D, _D), lambda j: (0, 0)),
        ],
        out_specs=pl.BlockSpec((_BLK, _D), lambda j: (j, 0)),
        out_shape=jax.ShapeDtypeStruct((_NP, _D), jnp.float32),
    )(S, S, hwp, dis, b, W)


def _tc_last_body(s0_ref, s1_ref, hwp_ref, dis_ref, b_ref, wo_ref, bo_ref,
                  out_ref):
    h = dis_ref[...] * (s0_ref[...] + s1_ref[...] + hwp_ref[...]) + b_ref[...]
    h = jnp.maximum(h, 0.0)
    z = jnp.dot(h, wo_ref[...], preferred_element_type=jnp.float32) + bo_ref[...]
    out_ref[...] = jax.nn.sigmoid(z)


def _tc_last(S, hwp, dis, b3, Wo, bo):
    return pl.pallas_call(
        _tc_last_body,
        grid=(_GRID,),
        in_specs=[
            pl.BlockSpec((_BLK, _D), lambda j: (j, 0)),
            pl.BlockSpec((_BLK, _D), lambda j: (j + _GRID, 0)),
            pl.BlockSpec((_BLK, _D), lambda j: (j, 0)),
            pl.BlockSpec((_BLK, _D), lambda j: (j, 0)),
            pl.BlockSpec((1, _D), lambda j: (0, 0)),
            pl.BlockSpec((_D, 1), lambda j: (0, 0)),
            pl.BlockSpec((1, 1), lambda j: (0, 0)),
        ],
        out_specs=pl.BlockSpec((_BLK, 1), lambda j: (j, 0)),
        out_shape=jax.ShapeDtypeStruct((_N, 1), jnp.float32),
    )(S, S, hwp, dis, b3, Wo, bo)


def kernel(x, edge_index, W1, b1, W2, b2, W3, b3, Wo, bo):
    src1 = edge_index[0]
    dst2 = edge_index[1].reshape(_NW, _NCH, _CH)
    zeros = jnp.zeros((_NP, _D), jnp.float32)
    ones = jnp.ones((_CH, _D), jnp.float32)

    deg = _sc_deg(dst2, zeros, ones)
    dis, hw1p = _tc_first(deg, x, W1)
    S1 = _sc_agg(hw1p, src1, dst2, zeros)
    hw2p = _tc_mid(S1, hw1p, dis, b1.reshape(1, _D), W2)
    S2 = _sc_agg(hw2p, src1, dst2, zeros)
    hw3p = _tc_mid(S2, hw2p, dis, b2.reshape(1, _D), W3)
    S3 = _sc_agg(hw3p, src1, dst2, zeros)
    return _tc_last(S3, hw3p, dis, b3.reshape(1, _D), Wo, bo.reshape(1, 1))
